# baseline (speedup 1.0000x reference)
import functools

import jax
import jax.numpy as jnp
from jax import lax
from jax.experimental import pallas as pl
from jax.experimental.pallas import tpu as pltpu
from jax.experimental.pallas import tpu_sc as plsc


_NC = 2
_NS = 16
_NW = _NC * _NS
_LANES = 16
_BSC = 1
_CH = 32
_BL = 1024


def _make_sc_add(b, lfull, ls, d):
    lpw = ls // _NW
    nch = lpw // _CH
    nsteps = nch * b
    nvec = d // _LANES
    mesh = plsc.VectorSubcoreMesh(core_axis_name="c", subcore_axis_name="s")

    @functools.partial(
        pl.kernel,
        out_type=jax.ShapeDtypeStruct((b * ls, d), jnp.float32),
        mesh=mesh,
        scratch_types=[
            pltpu.VMEM((_CH, d), jnp.float32),
            pltpu.VMEM((_CH, d), jnp.float32),
            pltpu.VMEM((_CH, d), jnp.float32),
            pltpu.VMEM((_CH, d), jnp.float32),
            pltpu.SemaphoreType.DMA,
            pltpu.SemaphoreType.DMA,
            pltpu.SemaphoreType.DMA,
            pltpu.SemaphoreType.DMA,
            pltpu.SemaphoreType.DMA,
            pltpu.SemaphoreType.DMA,
        ],
    )
    def sc_add(x_hbm, table_hbm, out_hbm,
               xb0, xb1, tb0, tb1, sx0, sx1, st0, st1, so0, so1):
        bufs = (xb0, xb1)
        tbufs = (tb0, tb1)
        sxs = (sx0, sx1)
        sts = (st0, st1)
        sos = (so0, so1)
        cid = lax.axis_index("c")
        sid = lax.axis_index("s")
        wid = cid * _NS + sid
        tbase = wid * lpw

        def t_slice(i):
            return table_hbm.at[pl.ds(tbase + i * _CH, _CH)]

        def x_slice(i, bi):
            return x_hbm.at[pl.ds(bi * lfull + tbase + i * _CH, _CH)]

        def o_slice(i, bi):
            return out_hbm.at[pl.ds(bi * ls + tbase + i * _CH, _CH)]

        pltpu.async_copy(t_slice(0), tbufs[0], sts[0])
        pltpu.async_copy(t_slice(1), tbufs[1], sts[1])
        pltpu.async_copy(x_slice(0, 0), bufs[0], sxs[0])

        def iter_body(i2, _):
            for ip in range(2):
                i = 2 * i2 + ip
                pltpu.make_async_copy(t_slice(i), tbufs[ip], sts[ip]).wait()
                for bi in range(b):
                    p = (ip * b + bi) % 2
                    s = i * b + bi
                    xb = bufs[p]
                    pltpu.make_async_copy(
                        x_slice(i, bi), xb, sxs[p]).wait()
                    nbi = (bi + 1) % b
                    ni = i + (1 if bi == b - 1 else 0)

                    @pl.when(s + 1 < nsteps)
                    def _():
                        @pl.when(s >= 1)
                        def _():
                            pltpu.make_async_copy(
                                bufs[1 - p], o_slice(ni, nbi),
                                sos[1 - p]).wait()
                        pltpu.async_copy(
                            x_slice(ni, nbi), bufs[1 - p], sxs[1 - p])

                    tb = tbufs[ip]

                    @plsc.parallel_loop(0, _CH, step=1)
                    def add_body(r, xb=xb, tb=tb):
                        for c in range(nvec):
                            plsc.addupdate(
                                xb.at[r].at[pl.ds(c * _LANES, _LANES)],
                                tb[r, pl.ds(c * _LANES, _LANES)])

                    pltpu.async_copy(xb, o_slice(i, bi), sos[p])

                @pl.when(i + 2 < nch)
                def _():
                    pltpu.async_copy(t_slice(i + 2), tbufs[ip], sts[ip])
            return 0

        lax.fori_loop(0, nch // 2, iter_body, 0)
        sp, sl = nsteps - 2, nsteps - 1
        pltpu.make_async_copy(
            bufs[sp % 2], o_slice(sp // b, sp % b), sos[sp % 2]).wait()
        pltpu.make_async_copy(
            bufs[sl % 2], o_slice(sl // b, sl % b), sos[sl % 2]).wait()

    return sc_add


def _tc_add_kernel(x_ref, t_ref, o_ref):
    o_ref[...] = x_ref[...] + t_ref[...]


def _tc_add(x, table):
    B, L, D = x.shape
    btc = B - _BSC
    grid = (L // _BL, btc)
    return pl.pallas_call(
        _tc_add_kernel,
        grid=grid,
        in_specs=[
            pl.BlockSpec((1, _BL, D), lambda i, bb: (_BSC + bb, i, 0)),
            pl.BlockSpec((_BL, D), lambda i, bb: (i, 0)),
        ],
        out_specs=pl.BlockSpec((1, _BL, D), lambda i, bb: (bb, i, 0)),
        out_shape=jax.ShapeDtypeStruct((btc, L, D), x.dtype),
    )(x, table)


def kernel(x, table):
    B, L, D = x.shape
    out_sc = _make_sc_add(_BSC, L, L, D)(x.reshape(B * L, D), table)
    out_tc = _tc_add(x, table)
    return jnp.concatenate([out_sc.reshape(_BSC, L, D), out_tc], axis=0)


# speedup vs baseline: 1.2998x; 1.2998x over previous
import functools

import jax
import jax.numpy as jnp
from jax import lax
from jax.experimental import pallas as pl
from jax.experimental.pallas import tpu as pltpu
from jax.experimental.pallas import tpu_sc as plsc


_NC = 2
_NS = 16
_NW = _NC * _NS
_LANES = 16
_BSC = 1
_CH = 32
_BL = 1024


def _make_sc_add(b, lfull, ls, d):
    lpw = ls // _NW
    nch = lpw // _CH
    nsteps = nch * b
    nvec = d // _LANES
    mesh = plsc.VectorSubcoreMesh(core_axis_name="c", subcore_axis_name="s")

    @functools.partial(
        pl.kernel,
        out_type=jax.ShapeDtypeStruct((b * ls, d), jnp.float32),
        mesh=mesh,
        scratch_types=[
            pltpu.VMEM((_CH, d), jnp.float32),
            pltpu.VMEM((_CH, d), jnp.float32),
            pltpu.VMEM((_CH, d), jnp.float32),
            pltpu.VMEM((_CH, d), jnp.float32),
            pltpu.SemaphoreType.DMA,
            pltpu.SemaphoreType.DMA,
            pltpu.SemaphoreType.DMA,
            pltpu.SemaphoreType.DMA,
            pltpu.SemaphoreType.DMA,
            pltpu.SemaphoreType.DMA,
        ],
    )
    def sc_add(x_hbm, table_hbm, out_hbm,
               xb0, xb1, tb0, tb1, sx0, sx1, st0, st1, so0, so1):
        bufs = (xb0, xb1)
        tbufs = (tb0, tb1)
        sxs = (sx0, sx1)
        sts = (st0, st1)
        sos = (so0, so1)
        cid = lax.axis_index("c")
        sid = lax.axis_index("s")
        wid = cid * _NS + sid
        tbase = wid * lpw

        def t_slice(i):
            return table_hbm.at[pl.ds(tbase + i * _CH, _CH)]

        def x_slice(i, bi):
            return x_hbm.at[pl.ds(bi * lfull + tbase + i * _CH, _CH)]

        def o_slice(i, bi):
            return out_hbm.at[pl.ds(bi * ls + tbase + i * _CH, _CH)]

        pltpu.async_copy(t_slice(0), tbufs[0], sts[0])
        pltpu.async_copy(t_slice(1), tbufs[1], sts[1])
        pltpu.async_copy(x_slice(0, 0), bufs[0], sxs[0])

        def iter_body(i2, _):
            for ip in range(2):
                i = 2 * i2 + ip
                pltpu.make_async_copy(t_slice(i), tbufs[ip], sts[ip]).wait()
                for bi in range(b):
                    p = (ip * b + bi) % 2
                    s = i * b + bi
                    xb = bufs[p]
                    pltpu.make_async_copy(
                        x_slice(i, bi), xb, sxs[p]).wait()
                    nbi = (bi + 1) % b
                    ni = i + (1 if bi == b - 1 else 0)

                    @pl.when(s + 1 < nsteps)
                    def _():
                        @pl.when(s >= 1)
                        def _():
                            pltpu.make_async_copy(
                                bufs[1 - p], o_slice(ni, nbi),
                                sos[1 - p]).wait()
                        pltpu.async_copy(
                            x_slice(ni, nbi), bufs[1 - p], sxs[1 - p])

                    tb = tbufs[ip]

                    @plsc.parallel_loop(0, _CH, step=1, unroll=2)
                    def add_body(r, xb=xb, tb=tb):
                        for c in range(nvec):
                            plsc.addupdate(
                                xb.at[r].at[pl.ds(c * _LANES, _LANES)],
                                tb[r, pl.ds(c * _LANES, _LANES)])

                    pltpu.async_copy(xb, o_slice(i, bi), sos[p])

                @pl.when(i + 2 < nch)
                def _():
                    pltpu.async_copy(t_slice(i + 2), tbufs[ip], sts[ip])
            return 0

        lax.fori_loop(0, nch // 2, iter_body, 0)
        sp, sl = nsteps - 2, nsteps - 1
        pltpu.make_async_copy(
            bufs[sp % 2], o_slice(sp // b, sp % b), sos[sp % 2]).wait()
        pltpu.make_async_copy(
            bufs[sl % 2], o_slice(sl // b, sl % b), sos[sl % 2]).wait()

    return sc_add


def kernel(x, table):
    B, L, D = x.shape
    out = _make_sc_add(B, L, L, D)(x.reshape(B * L, D), table)
    return out.reshape(B, L, D)


# speedup vs baseline: 1.5659x; 1.2047x over previous
import functools

import jax
import jax.numpy as jnp
from jax import lax
from jax.experimental import pallas as pl
from jax.experimental.pallas import tpu as pltpu
from jax.experimental.pallas import tpu_sc as plsc


_NC = 2
_NS = 16
_NW = _NC * _NS
_LANES = 16
_CH = 16
_NXB = 4


def _make_sc_add(b, lfull, ls, d):
    lpw = ls // _NW
    nch = lpw // _CH
    nsteps = nch * b
    nvec = d // _LANES
    mesh = plsc.VectorSubcoreMesh(core_axis_name="c", subcore_axis_name="s")

    @functools.partial(
        pl.kernel,
        out_type=jax.ShapeDtypeStruct((b * ls, d), jnp.float32),
        mesh=mesh,
        scratch_types=(
            [pltpu.VMEM((_CH, d), jnp.float32)] * (_NXB + 2)
            + [pltpu.SemaphoreType.DMA] * (2 * _NXB + 2)
        ),
    )
    def sc_add(x_hbm, table_hbm, out_hbm, *scratch):
        bufs = scratch[:_NXB]
        tbufs = scratch[_NXB:_NXB + 2]
        sxs = scratch[_NXB + 2:2 * _NXB + 2]
        sts = scratch[2 * _NXB + 2:2 * _NXB + 4]
        sos = scratch[2 * _NXB + 4:3 * _NXB + 4]
        cid = lax.axis_index("c")
        sid = lax.axis_index("s")
        wid = cid * _NS + sid
        tbase = wid * lpw

        def t_slice(i):
            return table_hbm.at[pl.ds(tbase + i * _CH, _CH)]

        def x_slice(i, bi):
            return x_hbm.at[pl.ds(bi * lfull + tbase + i * _CH, _CH)]

        def o_slice(i, bi):
            return out_hbm.at[pl.ds(bi * ls + tbase + i * _CH, _CH)]

        pltpu.async_copy(t_slice(0), tbufs[0], sts[0])
        pltpu.async_copy(t_slice(1), tbufs[1], sts[1])
        pltpu.async_copy(x_slice(0, 0), bufs[0], sxs[0])
        pltpu.async_copy(x_slice(0, 1), bufs[1], sxs[1])

        def iter_body(i2, _):
            for ip in range(2):
                i = 2 * i2 + ip
                pltpu.make_async_copy(t_slice(i), tbufs[ip], sts[ip]).wait()
                for bi in range(b):
                    p = bi % _NXB
                    s = i * b + bi
                    xb = bufs[p]
                    pltpu.make_async_copy(
                        x_slice(i, bi), xb, sxs[p]).wait()
                    np_ = (p + 2) % _NXB
                    nbi = (bi + 2) % b
                    ni = i + (1 if bi >= b - 2 else 0)

                    @pl.when(s + 2 < nsteps)
                    def _():
                        @pl.when(s >= 2)
                        def _():
                            pltpu.make_async_copy(
                                bufs[np_], o_slice(ni, nbi),
                                sos[np_]).wait()
                        pltpu.async_copy(
                            x_slice(ni, nbi), bufs[np_], sxs[np_])

                    tb = tbufs[ip]

                    @plsc.parallel_loop(0, _CH, step=1)
                    def add_body(r, xb=xb, tb=tb):
                        for c in range(nvec):
                            plsc.addupdate(
                                xb.at[r].at[pl.ds(c * _LANES, _LANES)],
                                tb[r, pl.ds(c * _LANES, _LANES)])

                    pltpu.async_copy(xb, o_slice(i, bi), sos[p])

                @pl.when(i + 2 < nch)
                def _():
                    pltpu.async_copy(t_slice(i + 2), tbufs[ip], sts[ip])
            return 0

        lax.fori_loop(0, nch // 2, iter_body, 0)
        for k in range(_NXB):
            sk = nsteps - _NXB + k
            pltpu.make_async_copy(
                bufs[sk % _NXB], o_slice(sk // b, sk % b),
                sos[sk % _NXB]).wait()

    return sc_add


def kernel(x, table):
    B, L, D = x.shape
    out = _make_sc_add(B, L, L, D)(x.reshape(B * L, D), table)
    return out.reshape(B, L, D)
